# baseline (speedup 1.0000x reference)
import functools

import jax
import jax.numpy as jnp
from jax import lax
from jax.experimental import pallas as pl
from jax.experimental.pallas import tpu as pltpu
from jax.experimental.pallas import tpu_sc as plsc

_P = 64
_K = 8
_N_IDX = _P * _K
_D = 2048
_NC = 2
_NS = 16
_NW = _NC * _NS
_L = 16
_CH = 16


def _build_sc_gather(R):
    rows_per_w = R // _NW
    n_chunks = rows_per_w // _CH
    mesh = plsc.VectorSubcoreMesh(core_axis_name="c", subcore_axis_name="s")

    @functools.partial(
        pl.kernel,
        mesh=mesh,
        out_type=jax.ShapeDtypeStruct((R * _N_IDX,), jnp.float32),
        scratch_types=[
            pltpu.VMEM((_N_IDX,), jnp.int32),
            pltpu.VMEM((_CH * _D,), jnp.float32),
            pltpu.VMEM((_CH * _N_IDX,), jnp.float32),
        ],
        compiler_params=pltpu.CompilerParams(needs_layout_passes=False),
    )
    def sc_gather(y_hbm, idx_hbm, out_hbm, idx_v, in_v, out_v):
        cid = lax.axis_index("c")
        sid = lax.axis_index("s")
        wid = sid * _NC + cid
        base = wid * rows_per_w
        pltpu.sync_copy(idx_hbm, idx_v)

        def chunk_body(c, carry):
            row0 = base + c * _CH
            pltpu.sync_copy(y_hbm.at[pl.ds(row0 * _D, _CH * _D)], in_v)

            def row_body(r, carry2):
                r_base = jnp.full((_L,), 0, jnp.int32) + r * _D
                for g in range(_N_IDX // _L):
                    cidx = idx_v[pl.ds(g * _L, _L)]
                    vals = plsc.load_gather(in_v, [r_base + cidx])
                    out_v[pl.ds(r * _N_IDX + g * _L, _L)] = vals
                return carry2

            lax.fori_loop(0, _CH, row_body, 0)
            pltpu.sync_copy(out_v, out_hbm.at[pl.ds(row0 * _N_IDX, _CH * _N_IDX)])
            return carry

        lax.fori_loop(0, n_chunks, chunk_body, 0)

    return sc_gather


def kernel(y, src_idx):
    B, T, D = y.shape
    R = B * T
    y_flat = y.reshape(R * D)
    idx_flat = src_idx.reshape(-1).astype(jnp.int32)
    out = _build_sc_gather(R)(y_flat, idx_flat)
    return out.reshape(B, T, _P, _K)


# speedup vs baseline: 1.1362x; 1.1362x over previous
import functools

import jax
import jax.numpy as jnp
from jax import lax
from jax.experimental import pallas as pl
from jax.experimental.pallas import tpu as pltpu
from jax.experimental.pallas import tpu_sc as plsc

_P = 64
_K = 8
_N_IDX = _P * _K
_D = 2048
_NC = 2
_NS = 16
_NW = _NC * _NS
_L = 16
_CH = 16
_NBUF = 2


def _build_sc_gather(R):
    rows_per_w = R // _NW
    n_chunks = rows_per_w // _CH
    n_groups = _N_IDX // _L
    mesh = plsc.VectorSubcoreMesh(core_axis_name="c", subcore_axis_name="s")

    @functools.partial(
        pl.kernel,
        mesh=mesh,
        out_type=jax.ShapeDtypeStruct((R * _N_IDX,), jnp.float32),
        scratch_types=[
            pltpu.VMEM((_N_IDX,), jnp.int32),
            pltpu.VMEM((_CH * _D,), jnp.float32),
            pltpu.VMEM((_CH * _D,), jnp.float32),
            pltpu.VMEM((_CH * _N_IDX,), jnp.float32),
            pltpu.VMEM((_CH * _N_IDX,), jnp.float32),
            pltpu.SemaphoreType.DMA,
            pltpu.SemaphoreType.DMA,
            pltpu.SemaphoreType.DMA,
            pltpu.SemaphoreType.DMA,
        ],
        compiler_params=pltpu.CompilerParams(needs_layout_passes=False),
    )
    def sc_gather(y_hbm, idx_hbm, out_hbm, idx_v, in0, in1, out0, out1,
                  isem0, isem1, osem0, osem1):
        cid = lax.axis_index("c")
        sid = lax.axis_index("s")
        wid = sid * _NC + cid
        base = wid * rows_per_w
        pltpu.sync_copy(idx_hbm, idx_v)

        ins = (in0, in1)
        outs = (out0, out1)
        isems = (isem0, isem1)
        osems = (osem0, osem1)

        def start_in(c, s):
            row0 = base + c * _CH
            pltpu.make_async_copy(
                y_hbm.at[pl.ds(row0 * _D, _CH * _D)], ins[s], isems[s]
            ).start()

        def wait_in(c, s):
            row0 = base + c * _CH
            pltpu.make_async_copy(
                y_hbm.at[pl.ds(row0 * _D, _CH * _D)], ins[s], isems[s]
            ).wait()

        def start_out(c, s):
            row0 = base + c * _CH
            pltpu.make_async_copy(
                outs[s], out_hbm.at[pl.ds(row0 * _N_IDX, _CH * _N_IDX)], osems[s]
            ).start()

        def wait_out(c, s):
            row0 = base + c * _CH
            pltpu.make_async_copy(
                outs[s], out_hbm.at[pl.ds(row0 * _N_IDX, _CH * _N_IDX)], osems[s]
            ).wait()

        cvecs = [idx_v[pl.ds(g * _L, _L)] for g in range(n_groups)]

        for s in range(_NBUF):
            start_in(s, s)

        def loop_body(t, carry):
            for s in range(_NBUF):
                c = _NBUF * t + s
                wait_in(c, s)

                @pl.when(c >= _NBUF)
                def _():
                    wait_out(c - _NBUF, s)

                for r in range(_CH):
                    rb = jnp.full((_L,), r * _D, jnp.int32)
                    for g in range(n_groups):
                        vals = plsc.load_gather(ins[s], [rb + cvecs[g]])
                        outs[s][pl.ds(r * _N_IDX + g * _L, _L)] = vals

                start_out(c, s)

                @pl.when(c + _NBUF < n_chunks)
                def _():
                    start_in(c + _NBUF, s)
            return carry

        lax.fori_loop(0, n_chunks // _NBUF, loop_body, 0)

        for s in range(_NBUF):
            wait_out(n_chunks - _NBUF + s, s)

    return sc_gather


def kernel(y, src_idx):
    B, T, D = y.shape
    R = B * T
    y_flat = y.reshape(R * D)
    idx_flat = src_idx.reshape(-1).astype(jnp.int32)
    out = _build_sc_gather(R)(y_flat, idx_flat)
    return out.reshape(B, T, _P, _K)


# speedup vs baseline: 2.8888x; 2.5425x over previous
import functools

import jax
import jax.numpy as jnp
from jax import lax
from jax.experimental import pallas as pl
from jax.experimental.pallas import tpu as pltpu
from jax.experimental.pallas import tpu_sc as plsc

_P = 64
_K = 8
_N_IDX = _P * _K
_D = 2048
_NC = 2
_NS = 16
_NW = _NC * _NS
_L = 16
_CH = 16
_NBUF = 2


def _build_sc_gather(R):
    rows_per_w = R // _NW
    n_chunks = rows_per_w // _CH
    n_groups = _N_IDX // _L
    mesh = plsc.VectorSubcoreMesh(core_axis_name="c", subcore_axis_name="s")

    @functools.partial(
        pl.kernel,
        mesh=mesh,
        out_type=jax.ShapeDtypeStruct((R, _N_IDX), jnp.float32),
        scratch_types=[
            pltpu.VMEM((_N_IDX,), jnp.int32),
            pltpu.VMEM((_CH, _D), jnp.float32),
            pltpu.VMEM((_CH, _D), jnp.float32),
            pltpu.VMEM((_CH, _N_IDX), jnp.float32),
            pltpu.VMEM((_CH, _N_IDX), jnp.float32),
            pltpu.SemaphoreType.DMA,
            pltpu.SemaphoreType.DMA,
            pltpu.SemaphoreType.DMA,
            pltpu.SemaphoreType.DMA,
        ],
        compiler_params=pltpu.CompilerParams(needs_layout_passes=False),
    )
    def sc_gather(y_hbm, idx_hbm, out_hbm, idx_v, in0, in1, out0, out1,
                  isem0, isem1, osem0, osem1):
        cid = lax.axis_index("c")
        sid = lax.axis_index("s")
        wid = sid * _NC + cid
        base = wid * rows_per_w
        pltpu.sync_copy(idx_hbm, idx_v)

        ins = (in0, in1)
        outs = (out0, out1)
        isems = (isem0, isem1)
        osems = (osem0, osem1)

        def start_in(c, s):
            row0 = base + c * _CH
            pltpu.make_async_copy(
                y_hbm.at[pl.ds(row0, _CH)], ins[s], isems[s]
            ).start()

        def wait_in(c, s):
            row0 = base + c * _CH
            pltpu.make_async_copy(
                y_hbm.at[pl.ds(row0, _CH)], ins[s], isems[s]
            ).wait()

        def start_out(c, s):
            row0 = base + c * _CH
            pltpu.make_async_copy(
                outs[s], out_hbm.at[pl.ds(row0, _CH)], osems[s]
            ).start()

        def wait_out(c, s):
            row0 = base + c * _CH
            pltpu.make_async_copy(
                outs[s], out_hbm.at[pl.ds(row0, _CH)], osems[s]
            ).wait()

        cvecs = [idx_v[pl.ds(g * _L, _L)] for g in range(n_groups)]
        lane = lax.iota(jnp.int32, _L)

        for s in range(_NBUF):
            start_in(s, s)

        def loop_body(t, carry):
            for s in range(_NBUF):
                c = _NBUF * t + s
                wait_in(c, s)

                @pl.when(c >= _NBUF)
                def _():
                    wait_out(c - _NBUF, s)

                for r in range(_CH):
                    rvec = jnp.full((_L,), r, jnp.int32)
                    for g in range(n_groups):
                        vals = plsc.load_gather(ins[s], [rvec, cvecs[g]])
                        plsc.store_scatter(outs[s], [rvec, lane + (g * _L)], vals)

                start_out(c, s)

                @pl.when(c + _NBUF < n_chunks)
                def _():
                    start_in(c + _NBUF, s)
            return carry

        lax.fori_loop(0, n_chunks // _NBUF, loop_body, 0)

        for s in range(_NBUF):
            wait_out(n_chunks - _NBUF + s, s)

    return sc_gather


def kernel(y, src_idx):
    B, T, D = y.shape
    R = B * T
    y_flat = y.reshape(R, D)
    idx_flat = src_idx.reshape(-1).astype(jnp.int32)
    out = _build_sc_gather(R)(y_flat, idx_flat)
    return out.reshape(B, T, _P, _K)


# speedup vs baseline: 3.3225x; 1.1501x over previous
import functools

import jax
import jax.numpy as jnp
from jax import lax
from jax.experimental import pallas as pl
from jax.experimental.pallas import tpu as pltpu
from jax.experimental.pallas import tpu_sc as plsc

_P = 64
_K = 8
_N_IDX = _P * _K
_D = 2048
_NC = 2
_NS = 16
_NW = _NC * _NS
_L = 16
_CH = 16
_NBUF = 2


def _build_sc_gather(R):
    rows_per_w = R // _NW
    n_chunks = rows_per_w // _CH
    n_groups = _N_IDX // _L
    mesh = plsc.VectorSubcoreMesh(core_axis_name="c", subcore_axis_name="s")

    @functools.partial(
        pl.kernel,
        mesh=mesh,
        out_type=jax.ShapeDtypeStruct((R, _N_IDX), jnp.float32),
        scratch_types=[
            pltpu.VMEM((_N_IDX,), jnp.int32),
            pltpu.VMEM((_CH, _D), jnp.float32),
            pltpu.VMEM((_CH, _D), jnp.float32),
            pltpu.VMEM((_CH, _N_IDX), jnp.float32),
            pltpu.VMEM((_CH, _N_IDX), jnp.float32),
            pltpu.SemaphoreType.DMA,
            pltpu.SemaphoreType.DMA,
            pltpu.SemaphoreType.DMA,
            pltpu.SemaphoreType.DMA,
        ],
        compiler_params=pltpu.CompilerParams(needs_layout_passes=False),
    )
    def sc_gather(y_hbm, idx_hbm, out_hbm, idx_v, in0, in1, out0, out1,
                  isem0, isem1, osem0, osem1):
        cid = lax.axis_index("c")
        sid = lax.axis_index("s")
        wid = sid * _NC + cid
        base = wid * rows_per_w
        pltpu.sync_copy(idx_hbm, idx_v)

        ins = (in0, in1)
        outs = (out0, out1)
        isems = (isem0, isem1)
        osems = (osem0, osem1)

        def start_in(c, s):
            row0 = base + c * _CH
            pltpu.make_async_copy(
                y_hbm.at[pl.ds(row0, _CH)], ins[s], isems[s]
            ).start()

        def wait_in(c, s):
            row0 = base + c * _CH
            pltpu.make_async_copy(
                y_hbm.at[pl.ds(row0, _CH)], ins[s], isems[s]
            ).wait()

        def start_out(c, s):
            row0 = base + c * _CH
            pltpu.make_async_copy(
                outs[s], out_hbm.at[pl.ds(row0, _CH)], osems[s]
            ).start()

        def wait_out(c, s):
            row0 = base + c * _CH
            pltpu.make_async_copy(
                outs[s], out_hbm.at[pl.ds(row0, _CH)], osems[s]
            ).wait()

        for s in range(_NBUF):
            start_in(s, s)

        def loop_body(t, carry):
            for s in range(_NBUF):
                c = _NBUF * t + s
                wait_in(c, s)

                @pl.when(c >= _NBUF)
                def _():
                    wait_out(c - _NBUF, s)

                for g in range(n_groups):
                    cidx = idx_v[pl.ds(g * _L, _L)]
                    for r in range(_CH):
                        rvec = jnp.full((_L,), r, jnp.int32)
                        vals = plsc.load_gather(ins[s], [rvec, cidx])
                        outs[s][r, pl.ds(g * _L, _L)] = vals

                start_out(c, s)

                @pl.when(c + _NBUF < n_chunks)
                def _():
                    start_in(c + _NBUF, s)
            return carry

        lax.fori_loop(0, n_chunks // _NBUF, loop_body, 0)

        for s in range(_NBUF):
            wait_out(n_chunks - _NBUF + s, s)

    return sc_gather


def kernel(y, src_idx):
    B, T, D = y.shape
    R = B * T
    y_flat = y.reshape(R, D)
    idx_flat = src_idx.reshape(-1).astype(jnp.int32)
    out = _build_sc_gather(R)(y_flat, idx_flat)
    return out.reshape(B, T, _P, _K)


# speedup vs baseline: 3.9000x; 1.1738x over previous
import functools

import jax
import jax.numpy as jnp
from jax import lax
from jax.experimental import pallas as pl
from jax.experimental.pallas import tpu as pltpu
from jax.experimental.pallas import tpu_sc as plsc

_P = 64
_K = 8
_N_IDX = _P * _K
_D = 2048
_NC = 2
_NS = 16
_NW = _NC * _NS
_L = 16
_CH = 16
_NBUF = 2


def _build_sc_gather(R):
    rows_per_w = R // _NW
    n_chunks = rows_per_w // _CH
    n_groups = _N_IDX // _L
    mesh = plsc.VectorSubcoreMesh(core_axis_name="c", subcore_axis_name="s")

    @functools.partial(
        pl.kernel,
        mesh=mesh,
        out_type=jax.ShapeDtypeStruct((R, _N_IDX), jnp.float32),
        scratch_types=[
            pltpu.VMEM((_N_IDX,), jnp.int32),
            pltpu.VMEM((_CH, _D), jnp.float32),
            pltpu.VMEM((_CH, _D), jnp.float32),
            pltpu.VMEM((_CH, _N_IDX), jnp.float32),
            pltpu.VMEM((_CH, _N_IDX), jnp.float32),
            pltpu.SemaphoreType.DMA,
            pltpu.SemaphoreType.DMA,
            pltpu.SemaphoreType.DMA,
            pltpu.SemaphoreType.DMA,
        ],
        compiler_params=pltpu.CompilerParams(needs_layout_passes=False),
    )
    def sc_gather(y_hbm, idx_hbm, out_hbm, idx_v, in0, in1, out0, out1,
                  isem0, isem1, osem0, osem1):
        cid = lax.axis_index("c")
        sid = lax.axis_index("s")
        wid = sid * _NC + cid
        base = wid * rows_per_w
        pltpu.sync_copy(idx_hbm, idx_v)

        ins = (in0, in1)
        outs = (out0, out1)
        isems = (isem0, isem1)
        osems = (osem0, osem1)

        def start_in(c, s):
            row0 = base + c * _CH
            pltpu.make_async_copy(
                y_hbm.at[pl.ds(row0, _CH)], ins[s], isems[s]
            ).start()

        def wait_in(c, s):
            row0 = base + c * _CH
            pltpu.make_async_copy(
                y_hbm.at[pl.ds(row0, _CH)], ins[s], isems[s]
            ).wait()

        def start_out(c, s):
            row0 = base + c * _CH
            pltpu.make_async_copy(
                outs[s], out_hbm.at[pl.ds(row0, _CH)], osems[s]
            ).start()

        def wait_out(c, s):
            row0 = base + c * _CH
            pltpu.make_async_copy(
                outs[s], out_hbm.at[pl.ds(row0, _CH)], osems[s]
            ).wait()

        for s in range(_NBUF):
            start_in(s, s)

        def loop_body(t, carry):
            for s in range(_NBUF):
                c = _NBUF * t + s
                wait_in(c, s)

                @pl.when(c >= _NBUF)
                def _():
                    wait_out(c - _NBUF, s)

                def do_row(r):
                    rvec = jnp.full((_L,), 0, jnp.int32) + r
                    for g in range(n_groups):
                        cidx = idx_v[pl.ds(g * _L, _L)]
                        vals = plsc.load_gather(ins[s], [rvec, cidx])
                        outs[s][r, pl.ds(g * _L, _L)] = vals

                plsc.parallel_loop(0, _CH, 1, unroll=4)(do_row)

                start_out(c, s)

                @pl.when(c + _NBUF < n_chunks)
                def _():
                    start_in(c + _NBUF, s)
            return carry

        lax.fori_loop(0, n_chunks // _NBUF, loop_body, 0)

        for s in range(_NBUF):
            wait_out(n_chunks - _NBUF + s, s)

    return sc_gather


def kernel(y, src_idx):
    B, T, D = y.shape
    R = B * T
    y_flat = y.reshape(R, D)
    idx_flat = src_idx.reshape(-1).astype(jnp.int32)
    out = _build_sc_gather(R)(y_flat, idx_flat)
    return out.reshape(B, T, _P, _K)
